# baseline (speedup 1.0000x reference)
import functools

import jax
import jax.numpy as jnp
from jax import lax
from jax.experimental import pallas as pl
from jax.experimental.pallas import tpu as pltpu
from jax.experimental.pallas import tpu_sc as plsc

B, C, N, K = 8, 16, 4096, 32
OUT = 64
RBF_K = 16
GAMMA = 10.0

BQ = 256
SUB, LANES = 32, 128
NCAND = 4
SMASK = 31

BQ2 = 512

BH = B // 2
TOTAL = BH * N * K
NW = 32
PER_W = TOTAL // NW
CH = 128
GSUB = 16
STEP = GSUB * CH
NIT = PER_W // STEP
ROWS_W = PER_W // CH


def _make_topk_block(bofs):
    def _topk_block(xt_q_ref, x_ref, dist_ref, gidx_ref):
        b = pl.program_id(0) + bofs
        xq = xt_q_ref[0]
        xb = x_ref[0]
        inner = jax.lax.dot(xq, xb, preferred_element_type=jnp.float32)
        xx_q = jnp.sum(xq * xq, axis=1, keepdims=True)
        xx_b = jnp.sum(xb * xb, axis=0, keepdims=True)
        d2 = jnp.maximum(xx_q + xx_b - 2.0 * inner, 1.2e-36)

        bits = jax.lax.bitcast_convert_type(d2, jnp.int32).reshape(BQ, SUB, LANES)
        s_iota = jax.lax.broadcasted_iota(jnp.int32, (BQ, SUB, LANES), 1)
        key3 = jax.lax.bitcast_convert_type(
            (bits & ~jnp.int32(SMASK)) | s_iota, jnp.float32)
        FINF = jnp.float32(jnp.inf)

        lane_iota = jax.lax.broadcasted_iota(jnp.int32, (BQ, LANES), 1)
        cands, cols = [], []
        for _ in range(NCAND):
            m = jnp.min(key3, axis=1)
            cands.append(m)
            ms = jax.lax.bitcast_convert_type(m, jnp.int32)
            cols.append(((ms & jnp.int32(SMASK)) * LANES
                         + lane_iota).astype(jnp.float32))
            key3 = jnp.where(key3 == m[:, None, :], FINF, key3)
        ck = jnp.concatenate(cands, axis=1)
        colarr = jnp.concatenate(cols, axis=1)

        keys_acc = jnp.full((BQ, LANES), FINF, jnp.float32)
        cols_acc = jnp.zeros((BQ, LANES), jnp.float32)
        for k in range(K):
            m = jnp.min(ck, axis=1, keepdims=True)
            eq = ck == m
            colv = jnp.min(jnp.where(eq, colarr, FINF), axis=1, keepdims=True)
            lane_is_k = lane_iota == k
            keys_acc = jnp.where(lane_is_k, m, keys_acc)
            cols_acc = jnp.where(lane_is_k, colv, cols_acc)
            ck = jnp.where(eq, FINF, ck)
        keys = keys_acc[:, :K]
        colsel = cols_acc[:, :K].astype(jnp.int32)

        d2sel = jax.lax.bitcast_convert_type(
            jax.lax.bitcast_convert_type(keys, jnp.int32) & ~jnp.int32(SMASK),
            jnp.float32)
        dist_ref[0] = jnp.sqrt(jnp.maximum(d2sel, 1e-12))
        gidx_ref[0] = colsel + b


    return _topk_block


def _sc_gather_body(idx_hbm, table_hbm, out_hbm, idx_v, rows_v, gsem):
    cid = lax.axis_index("c")
    sid = lax.axis_index("s")
    wid = sid * 2 + cid
    rowbase = wid * ROWS_W
    outbase = wid * PER_W

    def body(it, carry):
        pltpu.sync_copy(idx_hbm.at[pl.ds(rowbase + it * GSUB, GSUB)], idx_v)
        handles = [
            pltpu.async_copy(table_hbm.at[idx_v.at[g]],
                             rows_v.at[pl.ds(g * CH, CH)], gsem)
            for g in range(GSUB)
        ]
        for h in handles:
            h.wait()
        pltpu.sync_copy(rows_v, out_hbm.at[pl.ds(outbase + it * STEP, STEP)])
        return carry

    lax.fori_loop(0, NIT, body, 0)


@functools.cache
def _make_sc_gather():
    return pl.kernel(
        _sc_gather_body,
        out_type=jax.ShapeDtypeStruct((TOTAL, C), jnp.float32),
        mesh=plsc.VectorSubcoreMesh(core_axis_name="c", subcore_axis_name="s"),
        scratch_types=[
            pltpu.VMEM((GSUB, CH), jnp.int32),
            pltpu.VMEM((STEP, C), jnp.float32),
            pltpu.SemaphoreType.DMA,
        ],
        compiler_params=pltpu.CompilerParams(use_tc_tiling_on_sc=False),
    )


def _gather_neighbors(idx2d, flat):
    return _make_sc_gather()(idx2d, flat)


def _group_rows(a):
    parts = [a[:, 8 * j:8 * j + 8] for j in range(4)]
    return jnp.stack(parts, axis=1).reshape(4 * a.shape[0], 8)


def _ungroup_rows(a, bq):
    a3 = a.reshape(bq, 4, 8)
    return jnp.concatenate([a3[:, j, :] for j in range(4)], axis=1)


def _mlp_half(xq, dist, neigh_p, w1c_ref, w1n8_ref, w1r8_ref,
              w2blk_ref, updw8_ref, ublk_ref, nrblk_ref, bnw8_ref,
              bnb8_ref, cent8_ref, rep16_ref, bq):
    R = 4 * bq

    dist_g = _group_rows(dist)
    dist_rep = jax.lax.dot(dist_g, rep16_ref[...],
                           preferred_element_type=jnp.float32)
    diff = dist_rep - cent8_ref[0]
    rbf_p = jnp.clip(jnp.exp(-GAMMA * diff * diff), 1e-10, 1.0)

    ch = jax.lax.dot(xq, w1c_ref[...], preferred_element_type=jnp.float32)
    ch_t = jnp.tile(ch, (1, 8))
    nh_p = jax.lax.dot(neigh_p, w1n8_ref[...],
                       preferred_element_type=jnp.float32)
    rh_p = jax.lax.dot(rbf_p, w1r8_ref[...],
                       preferred_element_type=jnp.float32)
    h3 = (nh_p + rh_p).reshape(bq, 4, 512) + ch_t[:, None, :]
    h3 = jnp.where(h3 > 0, h3, 0.2 * h3)
    lsum = jax.lax.dot(h3.reshape(R, 512), w2blk_ref[...],
                       preferred_element_type=jnp.float32)
    logits = _ungroup_rows(lsum, bq)
    lmax = jnp.max(logits, axis=1, keepdims=True)
    ex = jnp.exp(logits - lmax)
    att = ex / jnp.sum(ex, axis=1, keepdims=True)

    att_rep = jax.lax.dot(_group_rows(att), rep16_ref[...],
                          preferred_element_type=jnp.float32)
    wn_p = att_rep * neigh_p
    upd_p = jax.lax.dot(wn_p, updw8_ref[...],
                        preferred_element_type=jnp.float32)
    upd_p = upd_p / jnp.sqrt(1.0 + 1e-5) * bnw8_ref[0] + bnb8_ref[0]
    upd_p = jnp.where(upd_p > 0, upd_p, 0.02 * upd_p)
    usum = jax.lax.dot(upd_p, ublk_ref[...],
                       preferred_element_type=jnp.float32)
    updm = jnp.sum(usum.reshape(bq, 4, OUT), axis=1) * (1.0 / K)

    rsum = jax.lax.dot(neigh_p, nrblk_ref[...],
                       preferred_element_type=jnp.float32)
    resm = jnp.sum(rsum.reshape(bq, 4, OUT), axis=1) * (1.0 / K)
    feat = updm + 0.1 * resm
    return feat.T


def _mlp_block(xt_q_ref, dist_ref, neigh_ref, w1c_ref, w1n8_ref, w1r8_ref,
               w2blk_ref, updw8_ref, ublk_ref, nrblk_ref, bnw8_ref,
               bnb8_ref, cent8_ref, rep16_ref, out_ref):
    H = BQ2 // 2
    ws = (w1c_ref, w1n8_ref, w1r8_ref, w2blk_ref, updw8_ref, ublk_ref,
          nrblk_ref, bnw8_ref, bnb8_ref, cent8_ref, rep16_ref)
    f0 = _mlp_half(xt_q_ref[0, :H], dist_ref[0, :H],
                   neigh_ref[:4 * H], *ws, bq=H)
    f1 = _mlp_half(xt_q_ref[0, H:], dist_ref[0, H:],
                   neigh_ref[4 * H:], *ws, bq=H)
    out_ref[0, :, :H] = f0
    out_ref[0, :, H:] = f1


def _half(xt_h, x_h, flat, bofs, wpack):
    (w1c, w1n8, w1r8, w2blk, updw8, ublk, nrblk, bnw8, bnb8, cent8,
     rep16) = wpack
    dist, gidx = pl.pallas_call(
        _make_topk_block(bofs),
        grid=(BH, N // BQ),
        in_specs=[
            pl.BlockSpec((1, BQ, C), lambda b, q: (b, q, 0)),
            pl.BlockSpec((1, C, N), lambda b, q: (b, 0, 0)),
        ],
        out_specs=[
            pl.BlockSpec((1, BQ, K), lambda b, q: (b, q, 0)),
            pl.BlockSpec((1, BQ, K), lambda b, q: (b, q, 0)),
        ],
        out_shape=[
            jax.ShapeDtypeStruct((BH, N, K), jnp.float32),
            jax.ShapeDtypeStruct((BH, N, K), jnp.int32),
        ],
    )(xt_h, x_h)

    idx2d = gidx.reshape(TOTAL // CH, CH)
    neigh_flat = _gather_neighbors(idx2d, flat)
    neigh2d = neigh_flat.reshape(TOTAL // 8, 8 * C)

    rep = lambda b, q: (0, 0)
    return pl.pallas_call(
        _mlp_block,
        grid=(BH, N // BQ2),
        in_specs=[
            pl.BlockSpec((1, BQ2, C), lambda b, q: (b, q, 0)),
            pl.BlockSpec((1, BQ2, K), lambda b, q: (b, q, 0)),
            pl.BlockSpec((4 * BQ2, 128),
                         lambda b, q: (b * (N // BQ2) + q, 0)),
            pl.BlockSpec((C, 64), rep),
            pl.BlockSpec((128, 512), rep),
            pl.BlockSpec((128, 512), rep),
            pl.BlockSpec((512, 8), rep),
            pl.BlockSpec((128, 512), rep),
            pl.BlockSpec((512, 64), rep),
            pl.BlockSpec((128, 64), rep),
            pl.BlockSpec((1, 512), rep),
            pl.BlockSpec((1, 512), rep),
            pl.BlockSpec((1, 128), rep),
            pl.BlockSpec((8, 128), rep),
        ],
        out_specs=pl.BlockSpec((1, OUT, BQ2), lambda b, q: (b, 0, q)),
        out_shape=jax.ShapeDtypeStruct((BH, OUT, N), jnp.float32),
    )(xt_h, dist, neigh2d, w1c, w1n8, w1r8, w2blk, updw8, ublk, nrblk,
      bnw8, bnb8, cent8, rep16)


def kernel(x, idx_base, att_w1, att_w2, upd_w, bn_w, bn_b, res_w, centers):
    del idx_base
    xt = jnp.transpose(x, (0, 2, 1))
    flat = xt.reshape(B * N, C)

    eye8 = jnp.eye(8, dtype=jnp.float32)
    w1c = att_w1[:, :C].T
    w1n = att_w1[:, C:2 * C].T
    w1r = att_w1[:, 2 * C:].T

    def kron8(w):
        a, b2 = w.shape
        return (eye8[:, None, :, None] * w[None, :, None, :]).reshape(
            8 * a, 8 * b2)

    wpack = (w1c, kron8(w1n), kron8(w1r), kron8(att_w2.T), kron8(upd_w.T),
             jnp.tile(jnp.eye(OUT, dtype=jnp.float32), (8, 1)),
             jnp.tile(res_w.T, (8, 1)),
             jnp.tile(bn_w, 8).reshape(1, 8 * OUT),
             jnp.tile(bn_b, 8).reshape(1, 8 * OUT),
             jnp.tile(centers, 8).reshape(1, 8 * RBF_K),
             kron8(jnp.ones((1, C), jnp.float32)))

    halves = [_half(xt[h * BH:(h + 1) * BH], x[h * BH:(h + 1) * BH],
                    flat, h * BH, wpack) for h in range(2)]
    return jnp.concatenate(halves, axis=0)


# speedup vs baseline: 1.0753x; 1.0753x over previous
import functools

import jax
import jax.numpy as jnp
from jax import lax
from jax.experimental import pallas as pl
from jax.experimental.pallas import tpu as pltpu
from jax.experimental.pallas import tpu_sc as plsc

B, C, N, K = 8, 16, 4096, 32
OUT = 64
RBF_K = 16
GAMMA = 10.0

BQ = 256
SUB, LANES = 32, 128
NCAND = 4
SMASK = 31

BQ2 = 512

BH = B // 2
TOTAL = BH * N * K
NW = 32
PER_W = TOTAL // NW
CH = 128
GSUB = 16
STEP = GSUB * CH
NIT = PER_W // STEP
ROWS_W = PER_W // CH


def _make_topk_block(bofs):
    def _topk_block(xt_q_ref, x_ref, dist_ref, gidx_ref):
        b = pl.program_id(0) + bofs
        xq = xt_q_ref[0]
        xb = x_ref[0]
        inner = jax.lax.dot(xq, xb, preferred_element_type=jnp.float32)
        xx_q = jnp.sum(xq * xq, axis=1, keepdims=True)
        xx_b = jnp.sum(xb * xb, axis=0, keepdims=True)
        d2 = jnp.maximum(xx_q + xx_b - 2.0 * inner, 1.2e-36)

        bits = jax.lax.bitcast_convert_type(d2, jnp.int32).reshape(BQ, SUB, LANES)
        s_iota = jax.lax.broadcasted_iota(jnp.int32, (BQ, SUB, LANES), 1)
        key3 = jax.lax.bitcast_convert_type(
            (bits & ~jnp.int32(SMASK)) | s_iota, jnp.float32)
        FINF = jnp.float32(jnp.inf)

        lane_iota = jax.lax.broadcasted_iota(jnp.int32, (BQ, LANES), 1)
        cands, cols = [], []
        for _ in range(NCAND):
            m = jnp.min(key3, axis=1)
            cands.append(m)
            ms = jax.lax.bitcast_convert_type(m, jnp.int32)
            cols.append(((ms & jnp.int32(SMASK)) * LANES
                         + lane_iota).astype(jnp.float32))
            key3 = jnp.where(key3 == m[:, None, :], FINF, key3)
        ck = jnp.concatenate(cands, axis=1)
        colarr = jnp.concatenate(cols, axis=1)

        keys_acc = jnp.full((BQ, LANES), FINF, jnp.float32)
        cols_acc = jnp.zeros((BQ, LANES), jnp.float32)
        for k in range(K):
            m = jnp.min(ck, axis=1, keepdims=True)
            eq = ck == m
            colv = jnp.min(jnp.where(eq, colarr, FINF), axis=1, keepdims=True)
            lane_is_k = lane_iota == k
            keys_acc = jnp.where(lane_is_k, m, keys_acc)
            cols_acc = jnp.where(lane_is_k, colv, cols_acc)
            ck = jnp.where(eq, FINF, ck)
        keys = keys_acc[:, :K]
        colsel = cols_acc[:, :K].astype(jnp.int32)

        d2sel = jax.lax.bitcast_convert_type(
            jax.lax.bitcast_convert_type(keys, jnp.int32) & ~jnp.int32(SMASK),
            jnp.float32)
        dist_ref[0] = jnp.sqrt(jnp.maximum(d2sel, 1e-12))
        gidx_ref[0] = colsel + b


    return _topk_block


def _sc_gather_body(idx_hbm, table_hbm, out_hbm, idx_v, rows_v, gsem):
    cid = lax.axis_index("c")
    sid = lax.axis_index("s")
    wid = sid * 2 + cid
    rowbase = wid * ROWS_W
    outbase = wid * PER_W

    def body(it, carry):
        pltpu.sync_copy(idx_hbm.at[pl.ds(rowbase + it * GSUB, GSUB)], idx_v)
        handles = [
            pltpu.async_copy(table_hbm.at[idx_v.at[g]],
                             rows_v.at[pl.ds(g * CH, CH)], gsem)
            for g in range(GSUB)
        ]
        for h in handles:
            h.wait()
        pltpu.sync_copy(rows_v, out_hbm.at[pl.ds(outbase + it * STEP, STEP)])
        return carry

    lax.fori_loop(0, NIT, body, 0)


@functools.cache
def _make_sc_gather():
    return pl.kernel(
        _sc_gather_body,
        out_type=jax.ShapeDtypeStruct((TOTAL, C), jnp.float32),
        mesh=plsc.VectorSubcoreMesh(core_axis_name="c", subcore_axis_name="s"),
        scratch_types=[
            pltpu.VMEM((GSUB, CH), jnp.int32),
            pltpu.VMEM((STEP, C), jnp.float32),
            pltpu.SemaphoreType.DMA,
        ],
        compiler_params=pltpu.CompilerParams(use_tc_tiling_on_sc=False),
    )


def _gather_neighbors(idx2d, flat):
    return _make_sc_gather()(idx2d, flat)


def _group_rows(a):
    parts = [a[:, 8 * j:8 * j + 8] for j in range(4)]
    return jnp.stack(parts, axis=1).reshape(4 * a.shape[0], 8)


def _mlp_half(xq, dist, neigh_p, w1c_ref, w1n8_ref, w1r8_ref,
              w2blk_ref, updw8_ref, ublk_ref, nrblk_ref, bnw8_ref,
              bnb8_ref, cent8_ref, rep16_ref, bq):
    R = 4 * bq

    dist_g = _group_rows(dist)
    dist_rep = jax.lax.dot(dist_g, rep16_ref[...],
                           preferred_element_type=jnp.float32)
    diff = dist_rep - cent8_ref[0]
    rbf_p = jnp.clip(jnp.exp(-GAMMA * diff * diff), 1e-10, 1.0)

    ch = jax.lax.dot(xq, w1c_ref[...], preferred_element_type=jnp.float32)
    ch_t = jnp.tile(ch, (1, 8))
    nh_p = jax.lax.dot(neigh_p, w1n8_ref[...],
                       preferred_element_type=jnp.float32)
    rh_p = jax.lax.dot(rbf_p, w1r8_ref[...],
                       preferred_element_type=jnp.float32)
    h3 = (nh_p + rh_p).reshape(bq, 4, 512) + ch_t[:, None, :]
    h3 = jnp.where(h3 > 0, h3, 0.2 * h3)
    lsum = jax.lax.dot(h3.reshape(R, 512), w2blk_ref[...],
                       preferred_element_type=jnp.float32)
    l3 = lsum.reshape(bq, 4, 8)
    lmax = jnp.max(jnp.max(l3, axis=2, keepdims=True), axis=1, keepdims=True)
    ex3 = jnp.exp(l3 - lmax)
    ssum = jnp.sum(jnp.sum(ex3, axis=2, keepdims=True), axis=1,
                   keepdims=True)
    recip = 1.0 / ssum

    ex_rep = jax.lax.dot(ex3.reshape(R, 8), rep16_ref[...],
                         preferred_element_type=jnp.float32)
    att_rep = (ex_rep.reshape(bq, 4, 128) * recip).reshape(R, 128)
    wn_p = att_rep * neigh_p
    upd_p = jax.lax.dot(wn_p, updw8_ref[...],
                        preferred_element_type=jnp.float32)
    upd_p = upd_p / jnp.sqrt(1.0 + 1e-5) * bnw8_ref[0] + bnb8_ref[0]
    upd_p = jnp.where(upd_p > 0, upd_p, 0.02 * upd_p)
    usum = jax.lax.dot(upd_p, ublk_ref[...],
                       preferred_element_type=jnp.float32)
    updm = jnp.sum(usum.reshape(bq, 4, OUT), axis=1) * (1.0 / K)

    rsum = jax.lax.dot(neigh_p, nrblk_ref[...],
                       preferred_element_type=jnp.float32)
    resm = jnp.sum(rsum.reshape(bq, 4, OUT), axis=1) * (1.0 / K)
    feat = updm + 0.1 * resm
    return feat.T


def _mlp_block(xt_q_ref, dist_ref, neigh_ref, w1c_ref, w1n8_ref, w1r8_ref,
               w2blk_ref, updw8_ref, ublk_ref, nrblk_ref, bnw8_ref,
               bnb8_ref, cent8_ref, rep16_ref, out_ref):
    H = BQ2 // 2
    ws = (w1c_ref, w1n8_ref, w1r8_ref, w2blk_ref, updw8_ref, ublk_ref,
          nrblk_ref, bnw8_ref, bnb8_ref, cent8_ref, rep16_ref)
    f0 = _mlp_half(xt_q_ref[0, :H], dist_ref[0, :H],
                   neigh_ref[:4 * H], *ws, bq=H)
    f1 = _mlp_half(xt_q_ref[0, H:], dist_ref[0, H:],
                   neigh_ref[4 * H:], *ws, bq=H)
    out_ref[0, :, :H] = f0
    out_ref[0, :, H:] = f1


def _half(xt_h, x_h, flat, bofs, wpack):
    (w1c, w1n8, w1r8, w2blk, updw8, ublk, nrblk, bnw8, bnb8, cent8,
     rep16) = wpack
    dist, gidx = pl.pallas_call(
        _make_topk_block(bofs),
        grid=(BH, N // BQ),
        in_specs=[
            pl.BlockSpec((1, BQ, C), lambda b, q: (b, q, 0)),
            pl.BlockSpec((1, C, N), lambda b, q: (b, 0, 0)),
        ],
        out_specs=[
            pl.BlockSpec((1, BQ, K), lambda b, q: (b, q, 0)),
            pl.BlockSpec((1, BQ, K), lambda b, q: (b, q, 0)),
        ],
        out_shape=[
            jax.ShapeDtypeStruct((BH, N, K), jnp.float32),
            jax.ShapeDtypeStruct((BH, N, K), jnp.int32),
        ],
    )(xt_h, x_h)

    idx2d = gidx.reshape(TOTAL // CH, CH)
    neigh_flat = _gather_neighbors(idx2d, flat)
    neigh2d = neigh_flat.reshape(TOTAL // 8, 8 * C)

    rep = lambda b, q: (0, 0)
    return pl.pallas_call(
        _mlp_block,
        grid=(BH, N // BQ2),
        in_specs=[
            pl.BlockSpec((1, BQ2, C), lambda b, q: (b, q, 0)),
            pl.BlockSpec((1, BQ2, K), lambda b, q: (b, q, 0)),
            pl.BlockSpec((4 * BQ2, 128),
                         lambda b, q: (b * (N // BQ2) + q, 0)),
            pl.BlockSpec((C, 64), rep),
            pl.BlockSpec((128, 512), rep),
            pl.BlockSpec((128, 512), rep),
            pl.BlockSpec((512, 8), rep),
            pl.BlockSpec((128, 512), rep),
            pl.BlockSpec((512, 64), rep),
            pl.BlockSpec((128, 64), rep),
            pl.BlockSpec((1, 512), rep),
            pl.BlockSpec((1, 512), rep),
            pl.BlockSpec((1, 128), rep),
            pl.BlockSpec((8, 128), rep),
        ],
        out_specs=pl.BlockSpec((1, OUT, BQ2), lambda b, q: (b, 0, q)),
        out_shape=jax.ShapeDtypeStruct((BH, OUT, N), jnp.float32),
    )(xt_h, dist, neigh2d, w1c, w1n8, w1r8, w2blk, updw8, ublk, nrblk,
      bnw8, bnb8, cent8, rep16)


def kernel(x, idx_base, att_w1, att_w2, upd_w, bn_w, bn_b, res_w, centers):
    del idx_base
    xt = jnp.transpose(x, (0, 2, 1))
    flat = xt.reshape(B * N, C)

    eye8 = jnp.eye(8, dtype=jnp.float32)
    w1c = att_w1[:, :C].T
    w1n = att_w1[:, C:2 * C].T
    w1r = att_w1[:, 2 * C:].T

    def kron8(w):
        a, b2 = w.shape
        return (eye8[:, None, :, None] * w[None, :, None, :]).reshape(
            8 * a, 8 * b2)

    wpack = (w1c, kron8(w1n), kron8(w1r), kron8(att_w2.T), kron8(upd_w.T),
             jnp.tile(jnp.eye(OUT, dtype=jnp.float32), (8, 1)),
             jnp.tile(res_w.T, (8, 1)),
             jnp.tile(bn_w, 8).reshape(1, 8 * OUT),
             jnp.tile(bn_b, 8).reshape(1, 8 * OUT),
             jnp.tile(centers, 8).reshape(1, 8 * RBF_K),
             kron8(jnp.ones((1, C), jnp.float32)))

    halves = [_half(xt[h * BH:(h + 1) * BH], x[h * BH:(h + 1) * BH],
                    flat, h * BH, wpack) for h in range(2)]
    return jnp.concatenate(halves, axis=0)
